# baseline (speedup 1.0000x reference)
import jax
import jax.numpy as jnp
from jax.experimental import pallas as pl
from jax.experimental.pallas import tpu as pltpu

_BLOCK_ROWS = 256


def _fill_diag_block(fill_ref, x_ref, o_ref):
    i = pl.program_id(0)
    o_ref[...] = x_ref[...]
    cols = pl.ds(i * _BLOCK_ROWS, _BLOCK_ROWS)
    sub = x_ref[:, cols]
    r = jax.lax.broadcasted_iota(jnp.int32, (_BLOCK_ROWS, _BLOCK_ROWS), 0)
    c = jax.lax.broadcasted_iota(jnp.int32, (_BLOCK_ROWS, _BLOCK_ROWS), 1)
    o_ref[:, cols] = jnp.where(r == c, fill_ref[0], sub)


def kernel(x, fill_value):
    n_rows, n_cols = x.shape
    fill = jnp.asarray(fill_value, x.dtype).reshape(1)
    return pl.pallas_call(
        _fill_diag_block,
        grid=(n_rows // _BLOCK_ROWS,),
        in_specs=[
            pl.BlockSpec(memory_space=pltpu.SMEM),
            pl.BlockSpec((_BLOCK_ROWS, n_cols), lambda i: (i, 0)),
        ],
        out_specs=pl.BlockSpec((_BLOCK_ROWS, n_cols), lambda i: (i, 0)),
        out_shape=jax.ShapeDtypeStruct(x.shape, x.dtype),
        compiler_params=pltpu.CompilerParams(
            dimension_semantics=("parallel",),
        ),
    )(fill, x)
